# baseline (speedup 1.0000x reference)
import jax
import jax.numpy as jnp
from jax import lax
from jax.experimental import pallas as pl
from jax.experimental.pallas import tpu as pltpu
from jax.experimental.pallas import tpu_sc as plsc

N = 10000
E = 320000
H = 128

NC = 2
NS = 16
NW = NC * NS
CHUNK = 128
NCH = 40
NROUND = 8
K0 = 2
E_PAD = NS * NROUND * NCH * CHUNK
N_PAD = 10240
RPT = N_PAD // NS

_MESH = plsc.VectorSubcoreMesh(
    core_axis_name="c", subcore_axis_name="s", num_cores=NC, num_subcores=NS
)


def _agg_body(tab_hbm, src_hbm, dst_hbm, zer_hbm, out_hbm,
              src_v, dst_v, buf_a, buf_b, acc_sh, sem_a, sem_b):
    cid = lax.axis_index("c")
    sid = lax.axis_index("s")

    pltpu.sync_copy(zer_hbm, buf_a)
    for r in range(RPT // CHUNK):
        pltpu.sync_copy(buf_a, acc_sh.at[pl.ds(sid * RPT + r * CHUNK, CHUNK)])
    plsc.subcore_barrier()

    def run_round(r):
        pltpu.sync_copy(src_hbm.at[sid, pl.ds(r * NCH, NCH)], src_v)
        pltpu.sync_copy(dst_hbm.at[sid, pl.ds(r * NCH, NCH)], dst_v)

        pltpu.async_copy(tab_hbm.at[src_v.at[0]], buf_a, sem_a)

        def pair(i, _):
            pltpu.async_copy(tab_hbm.at[src_v.at[2 * i + 1]], buf_b, sem_b)
            pltpu.make_async_copy(tab_hbm.at[src_v.at[2 * i]], buf_a, sem_a).wait()
            pltpu.sync_copy(buf_a, acc_sh.at[dst_v.at[2 * i]], add=True)
            pltpu.async_copy(tab_hbm.at[src_v.at[2 * i + 2]], buf_a, sem_a)
            pltpu.make_async_copy(tab_hbm.at[src_v.at[2 * i + 1]], buf_b, sem_b).wait()
            pltpu.sync_copy(buf_b, acc_sh.at[dst_v.at[2 * i + 1]], add=True)
            return 0

        lax.fori_loop(0, NCH // 2 - 1, pair, 0)

        last = NCH - 2
        pltpu.async_copy(tab_hbm.at[src_v.at[last + 1]], buf_b, sem_b)
        pltpu.make_async_copy(tab_hbm.at[src_v.at[last]], buf_a, sem_a).wait()
        pltpu.sync_copy(buf_a, acc_sh.at[dst_v.at[last]], add=True)
        pltpu.make_async_copy(tab_hbm.at[src_v.at[last + 1]], buf_b, sem_b).wait()
        pltpu.sync_copy(buf_b, acc_sh.at[dst_v.at[last + 1]], add=True)

    @pl.when(cid == 0)
    def _():
        for r in range(K0):
            run_round(r)

    @pl.when(cid == 1)
    def _():
        for r in range(K0, NROUND):
            run_round(r)

    plsc.subcore_barrier()

    for r in range(RPT // CHUNK):
        off = sid * RPT + r * CHUNK
        pltpu.sync_copy(acc_sh.at[pl.ds(off, CHUNK)], buf_a)
        pltpu.sync_copy(buf_a, out_hbm.at[cid, pl.ds(off, CHUNK)])


_agg_call = pl.kernel(
    _agg_body,
    out_type=jax.ShapeDtypeStruct((NC, N_PAD, H), jnp.float32),
    mesh=_MESH,
    scratch_types=[
        pltpu.VMEM((NCH, CHUNK), jnp.int32),
        pltpu.VMEM((NCH, CHUNK), jnp.int32),
        pltpu.VMEM((CHUNK, H), jnp.float32),
        pltpu.VMEM((CHUNK, H), jnp.float32),
        pltpu.VMEM_SHARED((N_PAD, H), jnp.float32),
        pltpu.SemaphoreType.DMA,
        pltpu.SemaphoreType.DMA,
    ],
)

BN = 1000
_GRID = (N // BN,)


def _dinv_block(cnt_ref):
    d = cnt_ref[0, :, 0] + cnt_ref[1, :, 0] + 1.0
    return lax.rsqrt(d)


def _tc1_body(cnt_ref, x_ref, w_ref, o_ref):
    dinv = _dinv_block(cnt_ref)
    s = jnp.dot(x_ref[...], w_ref[...], preferred_element_type=jnp.float32,
                precision=lax.Precision.HIGHEST)
    o_ref[...] = s * dinv[:, None]


def _tc2_body(cnt_ref, acc_ref, s_ref, b_ref, w_ref, o_ref):
    dinv = _dinv_block(cnt_ref)
    h = dinv[:, None] * (acc_ref[0] + acc_ref[1] + s_ref[...]) + b_ref[0]
    h = jnp.maximum(h, 0.0)
    o_ref[...] = jnp.dot(h, w_ref[...], preferred_element_type=jnp.float32,
                         precision=lax.Precision.HIGHEST) * dinv[:, None]


def _tc3_body(cnt_ref, acc_ref, s_ref, b_ref, o_ref):
    dinv = _dinv_block(cnt_ref)
    h = dinv[:, None] * (acc_ref[0] + acc_ref[1] + s_ref[...]) + b_ref[0]
    o_ref[...] = jnp.maximum(h, 0.0)


_CNT_SPEC = pl.BlockSpec((NC, BN, H), lambda i: (0, i, 0))
_ACC_SPEC = pl.BlockSpec((NC, BN, H), lambda i: (0, i, 0))
_ROW_SPEC = pl.BlockSpec((BN, H), lambda i: (i, 0))
_W_SPEC = pl.BlockSpec((H, H), lambda i: (0, 0))
_B_SPEC = pl.BlockSpec((1, H), lambda i: (0, 0))

_tc1 = pl.pallas_call(
    _tc1_body,
    grid=_GRID,
    in_specs=[_CNT_SPEC, _ROW_SPEC, _W_SPEC],
    out_specs=_ROW_SPEC,
    out_shape=jax.ShapeDtypeStruct((N, H), jnp.float32),
)

_tc2 = pl.pallas_call(
    _tc2_body,
    grid=_GRID,
    in_specs=[_CNT_SPEC, _ACC_SPEC, _ROW_SPEC, _B_SPEC, _W_SPEC],
    out_specs=_ROW_SPEC,
    out_shape=jax.ShapeDtypeStruct((N, H), jnp.float32),
)

_tc3 = pl.pallas_call(
    _tc3_body,
    grid=_GRID,
    in_specs=[_CNT_SPEC, _ACC_SPEC, _ROW_SPEC, _B_SPEC],
    out_specs=_ROW_SPEC,
    out_shape=jax.ShapeDtypeStruct((N, H), jnp.float32),
)


@jax.jit
def kernel(x, adj, W1, b1, W2, b2):
    pad = E_PAD - E
    src_r = jnp.concatenate(
        [adj[0], jnp.zeros((pad,), jnp.int32)]).reshape(NS, NROUND * NCH, CHUNK)
    dst_r = jnp.concatenate(
        [adj[1], jnp.full((pad,), N, jnp.int32)]).reshape(NS, NROUND * NCH, CHUNK)
    zer = jnp.zeros((CHUNK, H), jnp.float32)

    ones_tab = jnp.ones((N, H), jnp.float32)
    cnt = _agg_call(ones_tab, src_r, dst_r, zer)
    s1 = _tc1(cnt, x, W1)
    acc1 = _agg_call(s1, src_r, dst_r, zer)
    s2 = _tc2(cnt, acc1, s1, b1.reshape(1, H), W2)
    acc2 = _agg_call(s2, src_r, dst_r, zer)
    return _tc3(cnt, acc2, s2, b2.reshape(1, H))


# speedup vs baseline: 24.0318x; 24.0318x over previous
import jax
import jax.numpy as jnp
from jax import lax
from jax.experimental import pallas as pl
from jax.experimental.pallas import tpu as pltpu
from jax.experimental.pallas import tpu_sc as plsc

N = 10000
E = 320000
H = 128

NC = 2
NS = 16
NW = NC * NS
CHUNK = 128
NCH = 40
NROUND = 4
K0 = 2
E_PAD = NS * NROUND * NCH * CHUNK
N_PAD = 10240
RPT = N_PAD // NS

_MESH = plsc.VectorSubcoreMesh(
    core_axis_name="c", subcore_axis_name="s", num_cores=NC, num_subcores=NS
)


def _agg_body(tab_hbm, src_hbm, dst_hbm, zer_hbm, out_hbm,
              src_v, dst_v, buf_a, buf_b, acc_sh, sem_a, sem_b):
    cid = lax.axis_index("c")
    sid = lax.axis_index("s")

    pltpu.sync_copy(zer_hbm, buf_a)
    for r in range(RPT // CHUNK):
        pltpu.sync_copy(buf_a, acc_sh.at[pl.ds(sid * RPT + r * CHUNK, CHUNK)])
    plsc.subcore_barrier()

    def run_round(r):
        pltpu.sync_copy(src_hbm.at[sid, pl.ds(r * NCH, NCH)], src_v)
        pltpu.sync_copy(dst_hbm.at[sid, pl.ds(r * NCH, NCH)], dst_v)

        pltpu.async_copy(tab_hbm.at[src_v.at[0]], buf_a, sem_a)

        def pair(i, _):
            pltpu.async_copy(tab_hbm.at[src_v.at[2 * i + 1]], buf_b, sem_b)
            pltpu.make_async_copy(tab_hbm.at[src_v.at[2 * i]], buf_a, sem_a).wait()
            pltpu.sync_copy(buf_a, acc_sh.at[dst_v.at[2 * i]], add=True)
            pltpu.async_copy(tab_hbm.at[src_v.at[2 * i + 2]], buf_a, sem_a)
            pltpu.make_async_copy(tab_hbm.at[src_v.at[2 * i + 1]], buf_b, sem_b).wait()
            pltpu.sync_copy(buf_b, acc_sh.at[dst_v.at[2 * i + 1]], add=True)
            return 0

        lax.fori_loop(0, NCH // 2 - 1, pair, 0)

        last = NCH - 2
        pltpu.async_copy(tab_hbm.at[src_v.at[last + 1]], buf_b, sem_b)
        pltpu.make_async_copy(tab_hbm.at[src_v.at[last]], buf_a, sem_a).wait()
        pltpu.sync_copy(buf_a, acc_sh.at[dst_v.at[last]], add=True)
        pltpu.make_async_copy(tab_hbm.at[src_v.at[last + 1]], buf_b, sem_b).wait()
        pltpu.sync_copy(buf_b, acc_sh.at[dst_v.at[last + 1]], add=True)

    @pl.when(cid == 0)
    def _():
        for r in range(K0):
            run_round(r)

    @pl.when(cid == 1)
    def _():
        for r in range(K0, NROUND):
            run_round(r)

    plsc.subcore_barrier()

    for r in range(RPT // CHUNK):
        off = sid * RPT + r * CHUNK
        pltpu.sync_copy(acc_sh.at[pl.ds(off, CHUNK)], buf_a)
        pltpu.sync_copy(buf_a, out_hbm.at[cid, pl.ds(off, CHUNK)])


_agg_call = pl.kernel(
    _agg_body,
    out_type=jax.ShapeDtypeStruct((NC, N_PAD, H), jnp.float32),
    mesh=_MESH,
    scratch_types=[
        pltpu.VMEM((NCH, CHUNK), jnp.int32),
        pltpu.VMEM((NCH, CHUNK), jnp.int32),
        pltpu.VMEM((CHUNK, H), jnp.float32),
        pltpu.VMEM((CHUNK, H), jnp.float32),
        pltpu.VMEM_SHARED((N_PAD, H), jnp.float32),
        pltpu.SemaphoreType.DMA,
        pltpu.SemaphoreType.DMA,
    ],
)

BN = 1000
_GRID = (N // BN,)


def _dinv_block(cnt_ref):
    d = cnt_ref[0, :, 0] + cnt_ref[1, :, 0] + 1.0
    return lax.rsqrt(d)


def _tc1_body(cnt_ref, x_ref, w_ref, o_ref):
    dinv = _dinv_block(cnt_ref)
    s = jnp.dot(x_ref[...], w_ref[...], preferred_element_type=jnp.float32,
                precision=lax.Precision.HIGHEST)
    o_ref[...] = s * dinv[:, None]


def _tc2_body(cnt_ref, acc_ref, s_ref, b_ref, w_ref, o_ref):
    dinv = _dinv_block(cnt_ref)
    h = dinv[:, None] * (acc_ref[0] + acc_ref[1] + s_ref[...]) + b_ref[0]
    h = jnp.maximum(h, 0.0)
    o_ref[...] = jnp.dot(h, w_ref[...], preferred_element_type=jnp.float32,
                         precision=lax.Precision.HIGHEST) * dinv[:, None]


def _tc3_body(cnt_ref, acc_ref, s_ref, b_ref, o_ref):
    dinv = _dinv_block(cnt_ref)
    h = dinv[:, None] * (acc_ref[0] + acc_ref[1] + s_ref[...]) + b_ref[0]
    o_ref[...] = jnp.maximum(h, 0.0)


_CNT_SPEC = pl.BlockSpec((NC, BN, H), lambda i: (0, i, 0))
_ACC_SPEC = pl.BlockSpec((NC, BN, H), lambda i: (0, i, 0))
_ROW_SPEC = pl.BlockSpec((BN, H), lambda i: (i, 0))
_W_SPEC = pl.BlockSpec((H, H), lambda i: (0, 0))
_B_SPEC = pl.BlockSpec((1, H), lambda i: (0, 0))

_tc1 = pl.pallas_call(
    _tc1_body,
    grid=_GRID,
    in_specs=[_CNT_SPEC, _ROW_SPEC, _W_SPEC],
    out_specs=_ROW_SPEC,
    out_shape=jax.ShapeDtypeStruct((N, H), jnp.float32),
)

_tc2 = pl.pallas_call(
    _tc2_body,
    grid=_GRID,
    in_specs=[_CNT_SPEC, _ACC_SPEC, _ROW_SPEC, _B_SPEC, _W_SPEC],
    out_specs=_ROW_SPEC,
    out_shape=jax.ShapeDtypeStruct((N, H), jnp.float32),
)

_tc3 = pl.pallas_call(
    _tc3_body,
    grid=_GRID,
    in_specs=[_CNT_SPEC, _ACC_SPEC, _ROW_SPEC, _B_SPEC],
    out_specs=_ROW_SPEC,
    out_shape=jax.ShapeDtypeStruct((N, H), jnp.float32),
)


@jax.jit
def kernel(x, adj, W1, b1, W2, b2):
    pad = E_PAD - E
    src_r = jnp.concatenate(
        [adj[0], jnp.zeros((pad,), jnp.int32)]).reshape(NS, NROUND * NCH, CHUNK)
    pad_dst = N + (jnp.arange(pad, dtype=jnp.int32) % 128)
    dst_r = jnp.concatenate([adj[1], pad_dst]).reshape(NS, NROUND * NCH, CHUNK)
    zer = jnp.zeros((CHUNK, H), jnp.float32)

    ones_tab = jnp.ones((N, H), jnp.float32)
    cnt = _agg_call(ones_tab, src_r, dst_r, zer)
    s1 = _tc1(cnt, x, W1)
    acc1 = _agg_call(s1, src_r, dst_r, zer)
    s2 = _tc2(cnt, acc1, s1, b1.reshape(1, H), W2)
    acc2 = _agg_call(s2, src_r, dst_r, zer)
    return _tc3(cnt, acc2, s2, b2.reshape(1, H))


# speedup vs baseline: 31.7593x; 1.3216x over previous
import jax
import jax.numpy as jnp
from jax import lax
from jax.experimental import pallas as pl
from jax.experimental.pallas import tpu as pltpu
from jax.experimental.pallas import tpu_sc as plsc

N = 10000
E = 320000
H = 128

NC = 2
NS = 16
NW = NC * NS
CHUNK = 128
NCH = 40
NROUND = 4
K0 = 2
E_PAD = NS * NROUND * NCH * CHUNK
N_PAD = 10240
RPT = N_PAD // NS

_MESH = plsc.VectorSubcoreMesh(
    core_axis_name="c", subcore_axis_name="s", num_cores=NC, num_subcores=NS
)


def _agg_body(tab_hbm, src_hbm, dst_hbm, zer_hbm, out_hbm,
              src_v, dst_v, buf_a, buf_b, acc_sh, sem_a, sem_b):
    cid = lax.axis_index("c")
    sid = lax.axis_index("s")

    pltpu.sync_copy(zer_hbm, buf_a)
    for r in range(RPT // CHUNK):
        pltpu.sync_copy(buf_a, acc_sh.at[pl.ds(sid * RPT + r * CHUNK, CHUNK)])
    plsc.subcore_barrier()

    def run_round(r):
        pltpu.sync_copy(src_hbm.at[sid, pl.ds(r * NCH, NCH)], src_v)
        pltpu.sync_copy(dst_hbm.at[sid, pl.ds(r * NCH, NCH)], dst_v)

        pltpu.async_copy(tab_hbm.at[src_v.at[0]], buf_a, sem_a)

        def pair(i, _):
            pltpu.async_copy(tab_hbm.at[src_v.at[2 * i + 1]], buf_b, sem_b)
            pltpu.make_async_copy(tab_hbm.at[src_v.at[2 * i]], buf_a, sem_a).wait()
            pltpu.sync_copy(buf_a, acc_sh.at[dst_v.at[2 * i]], add=True)
            pltpu.async_copy(tab_hbm.at[src_v.at[2 * i + 2]], buf_a, sem_a)
            pltpu.make_async_copy(tab_hbm.at[src_v.at[2 * i + 1]], buf_b, sem_b).wait()
            pltpu.sync_copy(buf_b, acc_sh.at[dst_v.at[2 * i + 1]], add=True)
            return 0

        lax.fori_loop(0, NCH // 2 - 1, pair, 0)

        last = NCH - 2
        pltpu.async_copy(tab_hbm.at[src_v.at[last + 1]], buf_b, sem_b)
        pltpu.make_async_copy(tab_hbm.at[src_v.at[last]], buf_a, sem_a).wait()
        pltpu.sync_copy(buf_a, acc_sh.at[dst_v.at[last]], add=True)
        pltpu.make_async_copy(tab_hbm.at[src_v.at[last + 1]], buf_b, sem_b).wait()
        pltpu.sync_copy(buf_b, acc_sh.at[dst_v.at[last + 1]], add=True)

    @pl.when(cid == 0)
    def _():
        for r in range(K0):
            run_round(r)

    @pl.when(cid == 1)
    def _():
        for r in range(K0, NROUND):
            run_round(r)

    plsc.subcore_barrier()

    for r in range(RPT // CHUNK):
        off = sid * RPT + r * CHUNK
        pltpu.sync_copy(acc_sh.at[pl.ds(off, CHUNK)], buf_a)
        pltpu.sync_copy(buf_a, out_hbm.at[cid, pl.ds(off, CHUNK)])


def _cnt_body(one_hbm, dst_hbm, zer_hbm, out_hbm, dst_v, buf_a, acc_sh):
    cid = lax.axis_index("c")
    sid = lax.axis_index("s")

    pltpu.sync_copy(zer_hbm, buf_a)
    for r in range(RPT // CHUNK):
        pltpu.sync_copy(buf_a, acc_sh.at[pl.ds(sid * RPT + r * CHUNK, CHUNK)])
    plsc.subcore_barrier()

    pltpu.sync_copy(one_hbm, buf_a)

    def run_round(r):
        pltpu.sync_copy(dst_hbm.at[sid, pl.ds(r * NCH, NCH)], dst_v)

        def chunk(i, _):
            pltpu.sync_copy(buf_a, acc_sh.at[dst_v.at[i]], add=True)
            return 0

        lax.fori_loop(0, NCH, chunk, 0)

    @pl.when(cid == 0)
    def _():
        for r in range(K0):
            run_round(r)

    @pl.when(cid == 1)
    def _():
        for r in range(K0, NROUND):
            run_round(r)

    plsc.subcore_barrier()

    for r in range(RPT // CHUNK):
        off = sid * RPT + r * CHUNK
        pltpu.sync_copy(acc_sh.at[pl.ds(off, CHUNK)], buf_a)
        pltpu.sync_copy(buf_a, out_hbm.at[cid, pl.ds(off, CHUNK)])


_cnt_call = pl.kernel(
    _cnt_body,
    out_type=jax.ShapeDtypeStruct((NC, N_PAD, H), jnp.float32),
    mesh=_MESH,
    scratch_types=[
        pltpu.VMEM((NCH, CHUNK), jnp.int32),
        pltpu.VMEM((CHUNK, H), jnp.float32),
        pltpu.VMEM_SHARED((N_PAD, H), jnp.float32),
    ],
)


_agg_call = pl.kernel(
    _agg_body,
    out_type=jax.ShapeDtypeStruct((NC, N_PAD, H), jnp.float32),
    mesh=_MESH,
    scratch_types=[
        pltpu.VMEM((NCH, CHUNK), jnp.int32),
        pltpu.VMEM((NCH, CHUNK), jnp.int32),
        pltpu.VMEM((CHUNK, H), jnp.float32),
        pltpu.VMEM((CHUNK, H), jnp.float32),
        pltpu.VMEM_SHARED((N_PAD, H), jnp.float32),
        pltpu.SemaphoreType.DMA,
        pltpu.SemaphoreType.DMA,
    ],
)

BN = 1000
_GRID = (N // BN,)


def _dinv_block(cnt_ref):
    d = cnt_ref[0, :, 0] + cnt_ref[1, :, 0] + 1.0
    return lax.rsqrt(d)


def _tc1_body(cnt_ref, x_ref, w_ref, o_ref):
    dinv = _dinv_block(cnt_ref)
    s = jnp.dot(x_ref[...], w_ref[...], preferred_element_type=jnp.float32,
                precision=lax.Precision.HIGHEST)
    o_ref[...] = s * dinv[:, None]


def _tc2_body(cnt_ref, acc_ref, s_ref, b_ref, w_ref, o_ref):
    dinv = _dinv_block(cnt_ref)
    h = dinv[:, None] * (acc_ref[0] + acc_ref[1] + s_ref[...]) + b_ref[0]
    h = jnp.maximum(h, 0.0)
    o_ref[...] = jnp.dot(h, w_ref[...], preferred_element_type=jnp.float32,
                         precision=lax.Precision.HIGHEST) * dinv[:, None]


def _tc3_body(cnt_ref, acc_ref, s_ref, b_ref, o_ref):
    dinv = _dinv_block(cnt_ref)
    h = dinv[:, None] * (acc_ref[0] + acc_ref[1] + s_ref[...]) + b_ref[0]
    o_ref[...] = jnp.maximum(h, 0.0)


_CNT_SPEC = pl.BlockSpec((NC, BN, H), lambda i: (0, i, 0))
_ACC_SPEC = pl.BlockSpec((NC, BN, H), lambda i: (0, i, 0))
_ROW_SPEC = pl.BlockSpec((BN, H), lambda i: (i, 0))
_W_SPEC = pl.BlockSpec((H, H), lambda i: (0, 0))
_B_SPEC = pl.BlockSpec((1, H), lambda i: (0, 0))

_tc1 = pl.pallas_call(
    _tc1_body,
    grid=_GRID,
    in_specs=[_CNT_SPEC, _ROW_SPEC, _W_SPEC],
    out_specs=_ROW_SPEC,
    out_shape=jax.ShapeDtypeStruct((N, H), jnp.float32),
)

_tc2 = pl.pallas_call(
    _tc2_body,
    grid=_GRID,
    in_specs=[_CNT_SPEC, _ACC_SPEC, _ROW_SPEC, _B_SPEC, _W_SPEC],
    out_specs=_ROW_SPEC,
    out_shape=jax.ShapeDtypeStruct((N, H), jnp.float32),
)

_tc3 = pl.pallas_call(
    _tc3_body,
    grid=_GRID,
    in_specs=[_CNT_SPEC, _ACC_SPEC, _ROW_SPEC, _B_SPEC],
    out_specs=_ROW_SPEC,
    out_shape=jax.ShapeDtypeStruct((N, H), jnp.float32),
)


@jax.jit
def kernel(x, adj, W1, b1, W2, b2):
    pad = E_PAD - E
    src_r = jnp.concatenate(
        [adj[0], jnp.zeros((pad,), jnp.int32)]).reshape(NS, NROUND * NCH, CHUNK)
    pad_dst = N + (jnp.arange(pad, dtype=jnp.int32) % 128)
    dst_r = jnp.concatenate([adj[1], pad_dst]).reshape(NS, NROUND * NCH, CHUNK)
    zer = jnp.zeros((CHUNK, H), jnp.float32)

    one = jnp.ones((CHUNK, H), jnp.float32)
    cnt = _cnt_call(one, dst_r, zer)
    s1 = _tc1(cnt, x, W1)
    acc1 = _agg_call(s1, src_r, dst_r, zer)
    s2 = _tc2(cnt, acc1, s1, b1.reshape(1, H), W2)
    acc2 = _agg_call(s2, src_r, dst_r, zer)
    return _tc3(cnt, acc2, s2, b2.reshape(1, H))


# speedup vs baseline: 39.8743x; 1.2555x over previous
import jax
import jax.numpy as jnp
from jax import lax
from jax.experimental import pallas as pl
from jax.experimental.pallas import tpu as pltpu
from jax.experimental.pallas import tpu_sc as plsc

N = 10000
E = 320000
H = 128

NC = 2
NS = 16
NW = NC * NS
CHUNK = 128
NCH = 40
NROUND = 4
K0 = 3
E_PAD = NS * NROUND * NCH * CHUNK
N_PAD = 10240
RPT = N_PAD // NS

_MESH = plsc.VectorSubcoreMesh(
    core_axis_name="c", subcore_axis_name="s", num_cores=NC, num_subcores=NS
)


def _agg_body(tab_hbm, src_hbm, dst_hbm, zer_hbm, out_hbm,
              src_v, dst_v, buf_a, buf_b, acc_sh, sem_a, sem_b):
    cid = lax.axis_index("c")
    sid = lax.axis_index("s")

    pltpu.sync_copy(zer_hbm, buf_a)
    for r in range(RPT // CHUNK):
        pltpu.sync_copy(buf_a, acc_sh.at[pl.ds(sid * RPT + r * CHUNK, CHUNK)])
    plsc.subcore_barrier()

    def run_round(r):
        pltpu.sync_copy(src_hbm.at[sid, pl.ds(r * NCH, NCH)], src_v)
        pltpu.sync_copy(dst_hbm.at[sid, pl.ds(r * NCH, NCH)], dst_v)

        pltpu.async_copy(tab_hbm.at[src_v.at[0]], buf_a, sem_a)

        def pair(i, _):
            pltpu.async_copy(tab_hbm.at[src_v.at[2 * i + 1]], buf_b, sem_b)
            pltpu.make_async_copy(tab_hbm.at[src_v.at[2 * i]], buf_a, sem_a).wait()
            pltpu.sync_copy(buf_a, acc_sh.at[dst_v.at[2 * i]], add=True)
            pltpu.async_copy(tab_hbm.at[src_v.at[2 * i + 2]], buf_a, sem_a)
            pltpu.make_async_copy(tab_hbm.at[src_v.at[2 * i + 1]], buf_b, sem_b).wait()
            pltpu.sync_copy(buf_b, acc_sh.at[dst_v.at[2 * i + 1]], add=True)
            return 0

        lax.fori_loop(0, NCH // 2 - 1, pair, 0)

        last = NCH - 2
        pltpu.async_copy(tab_hbm.at[src_v.at[last + 1]], buf_b, sem_b)
        pltpu.make_async_copy(tab_hbm.at[src_v.at[last]], buf_a, sem_a).wait()
        pltpu.sync_copy(buf_a, acc_sh.at[dst_v.at[last]], add=True)
        pltpu.make_async_copy(tab_hbm.at[src_v.at[last + 1]], buf_b, sem_b).wait()
        pltpu.sync_copy(buf_b, acc_sh.at[dst_v.at[last + 1]], add=True)

    @pl.when(cid == 0)
    def _():
        for r in range(K0):
            run_round(r)

    @pl.when(cid == 1)
    def _():
        for r in range(K0, NROUND):
            run_round(r)

    plsc.subcore_barrier()

    for r in range(RPT // CHUNK):
        off = sid * RPT + r * CHUNK
        pltpu.sync_copy(acc_sh.at[pl.ds(off, CHUNK)], buf_a)
        pltpu.sync_copy(buf_a, out_hbm.at[cid, pl.ds(off, CHUNK)])


def _cnt_body(one_hbm, dst_hbm, zer_hbm, out_hbm, dst_v, buf_a, acc_sh):
    cid = lax.axis_index("c")
    sid = lax.axis_index("s")

    pltpu.sync_copy(zer_hbm, buf_a)
    for r in range(RPT // CHUNK):
        pltpu.sync_copy(buf_a, acc_sh.at[pl.ds(sid * RPT + r * CHUNK, CHUNK)])
    plsc.subcore_barrier()

    pltpu.sync_copy(one_hbm, buf_a)

    def run_round(r):
        pltpu.sync_copy(dst_hbm.at[sid, pl.ds(r * NCH, NCH)], dst_v)

        def chunk(i, _):
            pltpu.sync_copy(buf_a, acc_sh.at[dst_v.at[i]], add=True)
            return 0

        lax.fori_loop(0, NCH, chunk, 0)

    @pl.when(cid == 0)
    def _():
        for r in range(K0):
            run_round(r)

    @pl.when(cid == 1)
    def _():
        for r in range(K0, NROUND):
            run_round(r)

    plsc.subcore_barrier()

    for r in range(RPT // CHUNK):
        off = sid * RPT + r * CHUNK
        pltpu.sync_copy(acc_sh.at[pl.ds(off, CHUNK)], buf_a)
        pltpu.sync_copy(buf_a, out_hbm.at[cid, pl.ds(off, CHUNK)])


_cnt_call = pl.kernel(
    _cnt_body,
    out_type=jax.ShapeDtypeStruct((NC, N_PAD, H), jnp.float32),
    mesh=_MESH,
    scratch_types=[
        pltpu.VMEM((NCH, CHUNK), jnp.int32),
        pltpu.VMEM((CHUNK, H), jnp.float32),
        pltpu.VMEM_SHARED((N_PAD, H), jnp.float32),
    ],
)


_agg_call = pl.kernel(
    _agg_body,
    out_type=jax.ShapeDtypeStruct((NC, N_PAD, H), jnp.float32),
    mesh=_MESH,
    scratch_types=[
        pltpu.VMEM((NCH, CHUNK), jnp.int32),
        pltpu.VMEM((NCH, CHUNK), jnp.int32),
        pltpu.VMEM((CHUNK, H), jnp.float32),
        pltpu.VMEM((CHUNK, H), jnp.float32),
        pltpu.VMEM_SHARED((N_PAD, H), jnp.float32),
        pltpu.SemaphoreType.DMA,
        pltpu.SemaphoreType.DMA,
    ],
)

BN = 1000
_GRID = (N // BN,)


def _dinv_block(cnt_ref):
    d = cnt_ref[0, :, 0] + cnt_ref[1, :, 0] + 1.0
    return lax.rsqrt(d)


def _tc1_body(cnt_ref, x_ref, w_ref, o_ref):
    dinv = _dinv_block(cnt_ref)
    s = jnp.dot(x_ref[...], w_ref[...], preferred_element_type=jnp.float32,
                precision=lax.Precision.HIGHEST)
    o_ref[...] = s * dinv[:, None]


def _tc2_body(cnt_ref, acc_ref, s_ref, b_ref, w_ref, o_ref):
    dinv = _dinv_block(cnt_ref)
    h = dinv[:, None] * (acc_ref[0] + acc_ref[1] + s_ref[...]) + b_ref[0]
    h = jnp.maximum(h, 0.0)
    o_ref[...] = jnp.dot(h, w_ref[...], preferred_element_type=jnp.float32,
                         precision=lax.Precision.HIGHEST) * dinv[:, None]


def _tc3_body(cnt_ref, acc_ref, s_ref, b_ref, o_ref):
    dinv = _dinv_block(cnt_ref)
    h = dinv[:, None] * (acc_ref[0] + acc_ref[1] + s_ref[...]) + b_ref[0]
    o_ref[...] = jnp.maximum(h, 0.0)


_CNT_SPEC = pl.BlockSpec((NC, BN, H), lambda i: (0, i, 0))
_ACC_SPEC = pl.BlockSpec((NC, BN, H), lambda i: (0, i, 0))
_ROW_SPEC = pl.BlockSpec((BN, H), lambda i: (i, 0))
_W_SPEC = pl.BlockSpec((H, H), lambda i: (0, 0))
_B_SPEC = pl.BlockSpec((1, H), lambda i: (0, 0))

_tc1 = pl.pallas_call(
    _tc1_body,
    grid=_GRID,
    in_specs=[_CNT_SPEC, _ROW_SPEC, _W_SPEC],
    out_specs=_ROW_SPEC,
    out_shape=jax.ShapeDtypeStruct((N, H), jnp.float32),
)

_tc2 = pl.pallas_call(
    _tc2_body,
    grid=_GRID,
    in_specs=[_CNT_SPEC, _ACC_SPEC, _ROW_SPEC, _B_SPEC, _W_SPEC],
    out_specs=_ROW_SPEC,
    out_shape=jax.ShapeDtypeStruct((N, H), jnp.float32),
)

_tc3 = pl.pallas_call(
    _tc3_body,
    grid=_GRID,
    in_specs=[_CNT_SPEC, _ACC_SPEC, _ROW_SPEC, _B_SPEC],
    out_specs=_ROW_SPEC,
    out_shape=jax.ShapeDtypeStruct((N, H), jnp.float32),
)


@jax.jit
def kernel(x, adj, W1, b1, W2, b2):
    pad = E_PAD - E
    src_r = jnp.concatenate(
        [adj[0], jnp.zeros((pad,), jnp.int32)]).reshape(NS, NROUND * NCH, CHUNK)
    pad_dst = N + (jnp.arange(pad, dtype=jnp.int32) % 128)
    dst_r = jnp.concatenate([adj[1], pad_dst]).reshape(NS, NROUND * NCH, CHUNK)
    zer = jnp.zeros((CHUNK, H), jnp.float32)

    one = jnp.ones((CHUNK, H), jnp.float32)
    cnt = _cnt_call(one, dst_r, zer)
    s1 = _tc1(cnt, x, W1)
    acc1 = _agg_call(s1, src_r, dst_r, zer)
    s2 = _tc2(cnt, acc1, s1, b1.reshape(1, H), W2)
    acc2 = _agg_call(s2, src_r, dst_r, zer)
    return _tc3(cnt, acc2, s2, b2.reshape(1, H))
